# baseline (speedup 1.0000x reference)
import jax
import jax.numpy as jnp
from jax import lax
from jax.experimental import pallas as pl
from jax.experimental.pallas import tpu as pltpu
from jax.experimental.pallas import tpu_sc as plsc

_B = 16384
_D = 64
_NC = 2
_NS = 16
_NW = _NC * _NS
_BPW = _B // _NW
_HALF = _BPW // 2
_CHUNK = 128


def _body(ent_hbm, rel_hbm, heads_hbm, rels_hbm, tails_hbm, out_hbm,
          hidx, ridx, tidx, hp, rp, tp, e1_v, r_v, e2_v, out_v, sem):
    wid = lax.axis_index("s") * _NC + lax.axis_index("c")
    base = wid * _BPW

    pltpu.sync_copy(heads_hbm.at[pl.ds(base, _BPW)], hidx)
    pltpu.sync_copy(rels_hbm.at[pl.ds(base, _BPW)], ridx)
    pltpu.sync_copy(tails_hbm.at[pl.ds(base, _BPW)], tidx)

    for k in range(_BPW // 16):
        sl = pl.ds(k * 16, 16)
        hp[sl] = hidx[sl] >> 1
        rp[sl] = ridx[sl] >> 1
        tp[sl] = tidx[sl] >> 1

    lanes16 = lax.iota(jnp.int32, 16)
    bfly = [jnp.bitwise_xor(lanes16, sh) for sh in (8, 4, 2, 1)]
    dnums = lax.GatherDimensionNumbers(
        offset_dims=(), collapsed_slice_dims=(0,), start_index_map=(0,))

    def shuffle(v, idx):
        return lax.gather(v, idx[:, None], dnums, slice_sizes=(1,),
                          mode=lax.GatherScatterMode.PROMISE_IN_BOUNDS)

    def lanesum(v):
        for idx in bfly:
            v = v + shuffle(v, idx)
        return v

    ones16 = jnp.ones((16,), jnp.int32)

    for half in range(2):
        hbase = half * _HALF
        copies = []
        for k in range(_HALF // _CHUNK):
            isl = pl.ds(hbase + k * _CHUNK, _CHUNK)
            vsl = pl.ds(k * _CHUNK, _CHUNK)
            copies.append(pltpu.async_copy(ent_hbm.at[hp.at[isl]], e1_v.at[vsl], sem))
            copies.append(pltpu.async_copy(rel_hbm.at[rp.at[isl]], r_v.at[vsl], sem))
            copies.append(pltpu.async_copy(ent_hbm.at[tp.at[isl]], e2_v.at[vsl], sem))
        for c in copies:
            c.wait()

        def group(g, carry):
            row0 = g * 16
            hparf = (hidx[pl.ds(hbase + row0, 16)] & ones16).astype(jnp.float32)
            rparf = (ridx[pl.ds(hbase + row0, 16)] & ones16).astype(jnp.float32)
            tparf = (tidx[pl.ds(hbase + row0, 16)] & ones16).astype(jnp.float32)
            s = jnp.zeros((16,), jnp.float32)
            for j in range(16):
                row = row0 + j
                jv = jnp.full((16,), j, jnp.int32)
                ph = shuffle(hparf, jv)
                pr = shuffle(rparf, jv)
                pt = shuffle(tparf, jv)
                acc = jnp.zeros((16,), jnp.float32)
                for c in range(_D // 16):
                    lo = pl.ds(c * 16, 16)
                    hi = pl.ds(64 + c * 16, 16)
                    a1 = e1_v[row, lo]
                    a = a1 + ph * (e1_v[row, hi] - a1)
                    b1 = r_v[row, lo]
                    b = b1 + pr * (r_v[row, hi] - b1)
                    d1 = e2_v[row, lo]
                    d2 = d1 + pt * (e2_v[row, hi] - d1)
                    acc = acc + (a * b) * d2
                s = jnp.where(lanes16 == j, lanesum(acc), s)
            out_v[pl.ds(row0, 16)] = 1.0 / (1.0 + jnp.exp(-s))
            return carry

        lax.fori_loop(0, _HALF // 16, group, 0)
        pltpu.sync_copy(out_v.at[pl.ds(0, _HALF)],
                        out_hbm.at[pl.ds(base + hbase, _HALF)])


def kernel(entity_embedding, relation_embedding, heads, relations, tails):
    mesh = plsc.VectorSubcoreMesh(core_axis_name="c", subcore_axis_name="s")
    run = pl.kernel(
        _body,
        out_type=jax.ShapeDtypeStruct((_B,), jnp.float32),
        mesh=mesh,
        scratch_types=[
            pltpu.VMEM((_BPW,), jnp.int32),
            pltpu.VMEM((_BPW,), jnp.int32),
            pltpu.VMEM((_BPW,), jnp.int32),
            pltpu.VMEM((_BPW,), jnp.int32),
            pltpu.VMEM((_BPW,), jnp.int32),
            pltpu.VMEM((_BPW,), jnp.int32),
            pltpu.VMEM((_HALF, 2 * _D), jnp.float32),
            pltpu.VMEM((_HALF, 2 * _D), jnp.float32),
            pltpu.VMEM((_HALF, 2 * _D), jnp.float32),
            pltpu.VMEM((_HALF,), jnp.float32),
            pltpu.SemaphoreType.DMA,
        ],
    )
    return run(entity_embedding.reshape(500000, 128),
               relation_embedding.reshape(500, 128),
               heads.astype(jnp.int32), relations.astype(jnp.int32),
               tails.astype(jnp.int32))
